# baseline (speedup 1.0000x reference)
import functools

import jax
import jax.numpy as jnp
from jax import lax
from jax.experimental import pallas as pl
from jax.experimental.pallas import tpu as pltpu
from jax.experimental.pallas import tpu_sc as plsc

N = 10000
E = 160000
EPAD = 172032
NPAD = 10240
NAGG = 10112
B = 128
BN = 1000
NPTP = NPAD // 16
NPTA = NAGG // 16


def _sc_mesh():
    return plsc.VectorSubcoreMesh(core_axis_name="c", subcore_axis_name="s")


def _degrees(dst, w, zeros1):
    eps = EPAD // 2
    ept = eps // 16
    nit = ept // B

    @functools.partial(
        pl.kernel,
        mesh=_sc_mesh(),
        out_type=jax.ShapeDtypeStruct((2, NPAD), jnp.float32),
        scratch_types=[
            pltpu.VMEM((B,), jnp.int32),
            pltpu.VMEM((B,), jnp.float32),
            pltpu.VMEM_SHARED((NPAD,), jnp.float32),
        ],
    )
    def kern(dst_h, w_h, z_h, deg_h, idxv, wv, deg_sh):
        cid = lax.axis_index("c")
        sid = lax.axis_index("s")

        nsl = pl.ds(sid * NPTP, NPTP)
        pltpu.sync_copy(z_h.at[nsl], deg_sh.at[nsl])
        plsc.subcore_barrier()

        ebase = cid * eps + sid * ept

        def deg_body(i, _):
            off = ebase + i * B
            pltpu.sync_copy(dst_h.at[pl.ds(off, B)], idxv)
            pltpu.sync_copy(w_h.at[pl.ds(off, B)], wv)
            pltpu.sync_copy(wv, deg_sh.at[idxv], add=True)
            return 0
        lax.fori_loop(0, nit, deg_body, 0)
        plsc.subcore_barrier()

        pltpu.sync_copy(deg_sh.at[nsl], deg_h.at[cid, nsl])

    return kern(dst, w, zeros1)


def _rsqrt_tc(deg2):
    def body(d_ref, o_ref):
        d = d_ref[0] + d_ref[1]
        o_ref[...] = lax.rsqrt(jnp.maximum(d, 1e-12)).reshape(1, NPAD)

    return pl.pallas_call(
        body,
        grid=(1,),
        in_specs=[pl.BlockSpec((2, NPAD), lambda i: (0, 0))],
        out_specs=pl.BlockSpec((1, NPAD), lambda i: (0, 0)),
        out_shape=jax.ShapeDtypeStruct((1, NPAD), jnp.float32),
    )(deg2).reshape(NPAD)


def _edge_norm(src, dst, w, dinv):
    eps = EPAD // 2
    ept = eps // 16
    nit = ept // B

    @functools.partial(
        pl.kernel,
        mesh=_sc_mesh(),
        out_type=jax.ShapeDtypeStruct((EPAD,), jnp.float32),
        scratch_types=[
            pltpu.VMEM((B,), jnp.int32),
            pltpu.VMEM((B,), jnp.int32),
            pltpu.VMEM((B,), jnp.float32),
            pltpu.VMEM((B,), jnp.float32),
            pltpu.VMEM((B,), jnp.float32),
            pltpu.SemaphoreType.DMA,
        ],
    )
    def kern(src_h, dst_h, w_h, dinv_h, norm_h, idxa, idxb, wv, dsv, ddv, sem):
        cid = lax.axis_index("c")
        sid = lax.axis_index("s")
        ebase = cid * eps + sid * ept

        def norm_body(i, _):
            off = ebase + i * B
            pltpu.sync_copy(src_h.at[pl.ds(off, B)], idxa)
            pltpu.sync_copy(dst_h.at[pl.ds(off, B)], idxb)
            pltpu.sync_copy(w_h.at[pl.ds(off, B)], wv)
            pltpu.async_copy(dinv_h.at[idxa], dsv, sem).wait()
            pltpu.async_copy(dinv_h.at[idxb], ddv, sem).wait()

            def grp(g, _):
                sl = pl.ds(g * 16, 16)
                wv[sl] = wv[sl] * dsv[sl] * ddv[sl]
                return 0
            lax.fori_loop(0, B // 16, grp, 0)
            pltpu.sync_copy(wv, norm_h.at[pl.ds(off, B)])
            return 0
        lax.fori_loop(0, nit, norm_body, 0)

    return kern(src, dst, w, dinv)


def _aggregate(h_flat, src2, dst2, norm, C, fastc=1):
    SLAB = 3584
    nbs = SLAB // B

    @functools.partial(
        pl.kernel,
        mesh=_sc_mesh(),
        out_type=jax.ShapeDtypeStruct((2, NAGG, C * 128), jnp.float32),
        scratch_types=[
            pltpu.VMEM((SLAB,), jnp.int32),
            pltpu.VMEM((SLAB,), jnp.int32),
            pltpu.VMEM((2, 128), jnp.int32),
            pltpu.VMEM((2, 128), jnp.int32),
            pltpu.VMEM((SLAB + 16,), jnp.float32),
            pltpu.VMEM((B, 128), jnp.float32),
            pltpu.VMEM((B, 128), jnp.float32),
            pltpu.VMEM_SHARED((NAGG, 128), jnp.float32),
            pltpu.SemaphoreType.DMA,
            pltpu.SemaphoreType.DMA,
            pltpu.SemaphoreType.DMA,
            pltpu.SemaphoreType.DMA,
        ],
    )
    def kern(h_h, src_h, dst_h, norm_h, out_h,
             isrc, idst, ish, idr, nrm, rows0, rows1, acc,
             sem0, sem1, ssem0, ssem1):
        cid = lax.axis_index("c")
        sid = lax.axis_index("s")
        nslabs = jnp.where(cid == fastc, 2, 1)
        ebase = jnp.where(cid == fastc, 0, 32 * SLAB) + sid * (nslabs * SLAB)
        nbase = sid * NPTA

        rows = (rows0, rows1)
        sems = (sem0, sem1)
        ssems = (ssem0, ssem1)

        def gather(db):
            return pltpu.make_async_copy(h_h.at[ish.at[db]], rows[db], sems[db])

        def scatter(db):
            return pltpu.make_async_copy(
                rows[db], acc.at[idr.at[db]], ssems[db])

        def build_gidx(b, ring, coff):
            for j in range(8):
                ish[ring, pl.ds(j * 16, 16)] = (
                    isrc[pl.ds(b * B + j * 16, 16)] + coff)

        def chunk_body(c, _):
            coff = c * N

            def zs(i, _):
                for j in range(8):
                    rows0[i, pl.ds(j * 16, 16)] = jnp.zeros((16,), jnp.float32)
                return 0
            lax.fori_loop(0, B, zs, 0)
            for k in range(4):
                pltpu.async_copy(
                    rows0, acc.at[pl.ds(nbase + k * 128, 128), :], sem0)
            pltpu.async_copy(
                rows0.at[pl.ds(0, NPTA - 512), :],
                acc.at[pl.ds(nbase + 512, NPTA - 512), :], sem0)
            for k in range(4):
                pltpu.make_async_copy(
                    rows0, acc.at[pl.ds(nbase, 128), :], sem0).wait()
            pltpu.make_async_copy(
                rows0.at[pl.ds(0, NPTA - 512), :],
                acc.at[pl.ds(nbase + 512, NPTA - 512), :], sem0).wait()
            plsc.subcore_barrier()

            def slab_body(s, _):
                @pl.when(s < nslabs)
                def _():
                    sbase = ebase + s * SLAB
                    pltpu.sync_copy(src_h.at[pl.ds(sbase, SLAB)], isrc)
                    pltpu.sync_copy(dst_h.at[pl.ds(sbase, SLAB)], idst)
                    pltpu.sync_copy(norm_h.at[pl.ds(sbase, SLAB)],
                                    nrm.at[pl.ds(0, SLAB)])

                    build_gidx(0, 0, coff)
                    gather(0).start()

                    def pair_body(b2, _):
                        for db in range(2):
                            b = b2 * 2 + db
                            rbuf = rows[db]

                            @pl.when(b + 1 < nbs)
                            def _():
                                build_gidx(b + 1, 1 - db, coff)
                                gather(1 - db).start()
                            gather(db).wait()

                            def scale_grp(rg, _):
                                nv = nrm[pl.ds(b * B + rg * 16, 16)]
                                for l in range(16):
                                    r = rg * 16 + l
                                    s_ = nv[l]
                                    for j in range(8):
                                        sl = pl.ds(j * 16, 16)
                                        rbuf[r, sl] = rbuf[r, sl] * s_
                                return 0
                            lax.fori_loop(0, B // 16, scale_grp, 0)

                            for j in range(8):
                                idr[db, pl.ds(j * 16, 16)] = (
                                    idst[pl.ds(b * B + j * 16, 16)])
                            pltpu.sync_copy(rbuf, acc.at[idr.at[db]], add=True)
                        return 0
                    lax.fori_loop(0, nbs // 2, pair_body, 0)
                return 0
            lax.fori_loop(0, 2, slab_body, 0)
            plsc.subcore_barrier()

            pltpu.sync_copy(
                acc.at[pl.ds(nbase, NPTA), :],
                out_h.at[cid, pl.ds(nbase, NPTA), pl.ds(c * 128, 128)])
            return 0
        lax.fori_loop(0, C, chunk_body, 0)

    return kern(h_flat, src2, dst2, norm)


def _mm_in_agg(a, W, b):
    K = a.shape[2]
    M = W.shape[1]

    def body(a_ref, w_ref, b_ref, o_ref):
        x = a_ref[0] + a_ref[1]
        y = jnp.dot(x, w_ref[...], preferred_element_type=jnp.float32)
        o_ref[...] = jnp.maximum(y + b_ref[...], 0.0)

    return pl.pallas_call(
        body,
        grid=(N // BN,),
        in_specs=[
            pl.BlockSpec((2, BN, K), lambda i: (0, i, 0)),
            pl.BlockSpec((K, M), lambda i: (0, 0)),
            pl.BlockSpec((1, M), lambda i: (0, 0)),
        ],
        out_specs=pl.BlockSpec((BN, M), lambda i: (i, 0)),
        out_shape=jax.ShapeDtypeStruct((N, M), jnp.float32),
    )(a, W, b.reshape(1, M))


def _mm_chunks(h, W, C):
    K = h.shape[1]

    def body(h_ref, w_ref, o_ref):
        y = jnp.dot(h_ref[...], w_ref[...], preferred_element_type=jnp.float32)
        for j in range(C):
            o_ref[j] = y[:, j * 128:(j + 1) * 128]

    return pl.pallas_call(
        body,
        grid=(N // BN,),
        in_specs=[
            pl.BlockSpec((BN, K), lambda i: (i, 0)),
            pl.BlockSpec((K, C * 128), lambda i: (0, 0)),
        ],
        out_specs=pl.BlockSpec((C, BN, 128), lambda i: (0, i, 0)),
        out_shape=jax.ShapeDtypeStruct((C, N, 128), jnp.float32),
    )(h, W)


def _mm_relu_chunks(p, b_in, W, C):
    K = p.shape[2]

    def body(p_ref, b_ref, w_ref, o_ref):
        x = jnp.maximum(p_ref[0] + p_ref[1] + b_ref[...], 0.0)
        y = jnp.dot(x, w_ref[...], preferred_element_type=jnp.float32)
        for j in range(C):
            o_ref[j] = y[:, j * 128:(j + 1) * 128]

    return pl.pallas_call(
        body,
        grid=(N // BN,),
        in_specs=[
            pl.BlockSpec((2, BN, K), lambda i: (0, i, 0)),
            pl.BlockSpec((1, K), lambda i: (0, 0)),
            pl.BlockSpec((K, C * 128), lambda i: (0, 0)),
        ],
        out_specs=pl.BlockSpec((C, BN, 128), lambda i: (0, i, 0)),
        out_shape=jax.ShapeDtypeStruct((C, N, 128), jnp.float32),
    )(p, b_in.reshape(1, K), W)


def _mm_final(p, b_in, W, b_out):
    K = p.shape[2]
    M = W.shape[1]

    def body(p_ref, bi_ref, w_ref, bo_ref, o_ref):
        x = jnp.maximum(p_ref[0] + p_ref[1] + bi_ref[...], 0.0)
        y = jnp.dot(x, w_ref[...], preferred_element_type=jnp.float32)
        o_ref[...] = y + bo_ref[...]

    return pl.pallas_call(
        body,
        grid=(N // BN,),
        in_specs=[
            pl.BlockSpec((2, BN, K), lambda i: (0, i, 0)),
            pl.BlockSpec((1, K), lambda i: (0, 0)),
            pl.BlockSpec((K, M), lambda i: (0, 0)),
            pl.BlockSpec((1, M), lambda i: (0, 0)),
        ],
        out_specs=pl.BlockSpec((BN, M), lambda i: (i, 0)),
        out_shape=jax.ShapeDtypeStruct((N, M), jnp.float32),
    )(p, b_in.reshape(1, K), W, b_out.reshape(1, M))


def kernel(x, edge_index, edge_attr, W1, b1, W2, b2, W3, b3, W4, b4, W5, b5, Wc, bc):
    src = edge_index[0].astype(jnp.int32)
    dst = edge_index[1].astype(jnp.int32)
    loop_idx = jnp.arange(N, dtype=jnp.int32)
    pad = EPAD - (E + N)
    src_f = jnp.concatenate([src, loop_idx, jnp.zeros((pad,), jnp.int32)])
    dst_f = jnp.concatenate([dst, loop_idx, jnp.zeros((pad,), jnp.int32)])
    w_f = jnp.concatenate([
        edge_attr.astype(jnp.float32),
        jnp.ones((N,), jnp.float32),
        jnp.zeros((pad,), jnp.float32),
    ])

    deg2 = _degrees(dst_f, w_f, jnp.zeros((NPAD,), jnp.float32))
    dinv = _rsqrt_tc(deg2)
    norm = _edge_norm(src_f, dst_f, w_f, dinv)

    a1 = _aggregate(x, src_f, dst_f, norm, 1)
    h1 = _mm_in_agg(a1, W1, b1)

    t2 = _mm_chunks(h1, W2, 8)
    a2 = _aggregate(t2.reshape(8 * N, 128), src_f, dst_f, norm, 8)

    t3 = _mm_relu_chunks(a2, b2, W3, 4)
    a3 = _aggregate(t3.reshape(4 * N, 128), src_f, dst_f, norm, 4)

    t4 = _mm_relu_chunks(a3, b3, W4, 1)
    a4 = _aggregate(t4.reshape(N, 128), src_f, dst_f, norm, 1)

    W5p = jnp.pad(W5, ((0, 0), (0, 64)))
    b5p = jnp.pad(b5, (0, 64))
    t5 = _mm_relu_chunks(a4, b4, W5p, 1)
    a5 = _aggregate(t5.reshape(N, 128), src_f, dst_f, norm, 1)

    Wcp = jnp.pad(Wc, ((0, 64), (0, 125)))
    bcp = jnp.pad(bc, (0, 125))
    out = _mm_final(a5, b5p, Wcp, bcp)
    return out[:, :3]


# speedup vs baseline: 1.0599x; 1.0599x over previous
import functools

import jax
import jax.numpy as jnp
from jax import lax
from jax.experimental import pallas as pl
from jax.experimental.pallas import tpu as pltpu
from jax.experimental.pallas import tpu_sc as plsc

N = 10000
E = 160000
EPAD = 172032
NPAD = 10240
NAGG = 10112
B = 128
BN = 1000
NPTP = NPAD // 16
NPTA = NAGG // 16


def _sc_mesh():
    return plsc.VectorSubcoreMesh(core_axis_name="c", subcore_axis_name="s")


def _degrees(dst, w, zeros1):
    eps = EPAD // 2
    ept = eps // 16
    nit = ept // B

    @functools.partial(
        pl.kernel,
        mesh=_sc_mesh(),
        out_type=jax.ShapeDtypeStruct((2, NPAD), jnp.float32),
        scratch_types=[
            pltpu.VMEM((ept,), jnp.int32),
            pltpu.VMEM((ept,), jnp.float32),
            pltpu.VMEM((ept // B, 128), jnp.int32),
            pltpu.VMEM_SHARED((NPAD,), jnp.float32),
            pltpu.SemaphoreType.DMA,
        ],
    )
    def kern(dst_h, w_h, z_h, deg_h, dflat, wflat, idx2, deg_sh, sem):
        cid = lax.axis_index("c")
        sid = lax.axis_index("s")

        nsl = pl.ds(sid * NPTP, NPTP)
        pltpu.sync_copy(z_h.at[nsl], deg_sh.at[nsl])

        ebase = cid * eps + sid * ept
        pltpu.sync_copy(dst_h.at[pl.ds(ebase, ept)], dflat)
        pltpu.sync_copy(w_h.at[pl.ds(ebase, ept)], wflat)

        def expand(i, _):
            for j in range(8):
                idx2[i, pl.ds(j * 16, 16)] = dflat[pl.ds(i * B + j * 16, 16)]
            return 0
        lax.fori_loop(0, nit, expand, 0)
        plsc.subcore_barrier()

        def fire(b, _):
            pltpu.async_copy(
                wflat.at[pl.ds(b * B, B)], deg_sh.at[idx2.at[b]], sem, add=True)
            return 0
        lax.fori_loop(0, nit, fire, 0)

        def drain(b, _):
            pltpu.make_async_copy(
                wflat.at[pl.ds(b * B, B)], deg_sh.at[idx2.at[b]], sem).wait()
            return 0
        lax.fori_loop(0, nit, drain, 0)
        plsc.subcore_barrier()

        pltpu.sync_copy(deg_sh.at[nsl], deg_h.at[cid, nsl])

    return kern(dst, w, zeros1)


def _rsqrt_tc(deg2):
    def body(d_ref, o_ref):
        d = d_ref[0] + d_ref[1]
        o_ref[...] = lax.rsqrt(jnp.maximum(d, 1e-12)).reshape(1, NPAD)

    return pl.pallas_call(
        body,
        grid=(1,),
        in_specs=[pl.BlockSpec((2, NPAD), lambda i: (0, 0))],
        out_specs=pl.BlockSpec((1, NPAD), lambda i: (0, 0)),
        out_shape=jax.ShapeDtypeStruct((1, NPAD), jnp.float32),
    )(deg2).reshape(NPAD)


def _edge_norm(src, dst, w, dinv):
    eps = EPAD // 2
    ept = eps // 16
    nit = ept // B

    @functools.partial(
        pl.kernel,
        mesh=_sc_mesh(),
        out_type=jax.ShapeDtypeStruct((EPAD,), jnp.float32),
        scratch_types=[
            pltpu.VMEM((ept,), jnp.int32),
            pltpu.VMEM((ept,), jnp.int32),
            pltpu.VMEM((ept,), jnp.float32),
            pltpu.VMEM((ept,), jnp.float32),
            pltpu.VMEM((ept,), jnp.float32),
            pltpu.SemaphoreType.DMA,
        ],
    )
    def kern(src_h, dst_h, w_h, dinv_h, norm_h, sflat, dflat, wflat, dsv, ddv, sem):
        cid = lax.axis_index("c")
        sid = lax.axis_index("s")
        ebase = cid * eps + sid * ept

        pltpu.sync_copy(src_h.at[pl.ds(ebase, ept)], sflat)
        pltpu.sync_copy(dst_h.at[pl.ds(ebase, ept)], dflat)
        pltpu.sync_copy(w_h.at[pl.ds(ebase, ept)], wflat)

        def fire(b, _):
            sl = pl.ds(b * B, B)
            pltpu.async_copy(dinv_h.at[sflat.at[sl]], dsv.at[sl], sem)
            pltpu.async_copy(dinv_h.at[dflat.at[sl]], ddv.at[sl], sem)
            return 0
        lax.fori_loop(0, nit, fire, 0)

        def drain(b, _):
            sl = pl.ds(b * B, B)
            pltpu.make_async_copy(dinv_h.at[sflat.at[sl]], dsv.at[sl], sem).wait()
            pltpu.make_async_copy(dinv_h.at[dflat.at[sl]], ddv.at[sl], sem).wait()
            return 0
        lax.fori_loop(0, nit, drain, 0)

        def grp(g, _):
            sl = pl.ds(g * 16, 16)
            wflat[sl] = wflat[sl] * dsv[sl] * ddv[sl]
            return 0
        lax.fori_loop(0, ept // 16, grp, 0)
        pltpu.sync_copy(wflat, norm_h.at[pl.ds(ebase, ept)])

    return kern(src, dst, w, dinv)


def _aggregate(h_flat, src2, dst2, norm, C, fastc=0):
    SLAB = 3584
    nbs = SLAB // B

    @functools.partial(
        pl.kernel,
        mesh=_sc_mesh(),
        out_type=jax.ShapeDtypeStruct((2, NAGG, C * 128), jnp.float32),
        scratch_types=[
            pltpu.VMEM((SLAB,), jnp.int32),
            pltpu.VMEM((SLAB,), jnp.int32),
            pltpu.VMEM((2, 128), jnp.int32),
            pltpu.VMEM((2, 128), jnp.int32),
            pltpu.VMEM((SLAB + 16,), jnp.float32),
            pltpu.VMEM((B, 128), jnp.float32),
            pltpu.VMEM((B, 128), jnp.float32),
            pltpu.VMEM_SHARED((NAGG, 128), jnp.float32),
            pltpu.SemaphoreType.DMA,
            pltpu.SemaphoreType.DMA,
            pltpu.SemaphoreType.DMA,
            pltpu.SemaphoreType.DMA,
        ],
    )
    def kern(h_h, src_h, dst_h, norm_h, out_h,
             isrc, idst, ish, idr, nrm, rows0, rows1, acc,
             sem0, sem1, ssem0, ssem1):
        cid = lax.axis_index("c")
        sid = lax.axis_index("s")
        nslabs = jnp.where(cid == fastc, 2, 1)
        ebase = jnp.where(cid == fastc, 0, 32 * SLAB) + sid * (nslabs * SLAB)
        nbase = sid * NPTA

        rows = (rows0, rows1)
        sems = (sem0, sem1)
        ssems = (ssem0, ssem1)

        def gather(db):
            return pltpu.make_async_copy(h_h.at[ish.at[db]], rows[db], sems[db])

        def scatter(db):
            return pltpu.make_async_copy(
                rows[db], acc.at[idr.at[db]], ssems[db])

        def build_gidx(b, ring, coff):
            for j in range(8):
                ish[ring, pl.ds(j * 16, 16)] = (
                    isrc[pl.ds(b * B + j * 16, 16)] + coff)

        def chunk_body(c, _):
            coff = c * N

            def zs(i, _):
                for j in range(8):
                    rows0[i, pl.ds(j * 16, 16)] = jnp.zeros((16,), jnp.float32)
                return 0
            lax.fori_loop(0, B, zs, 0)
            for k in range(4):
                pltpu.async_copy(
                    rows0, acc.at[pl.ds(nbase + k * 128, 128), :], sem0)
            pltpu.async_copy(
                rows0.at[pl.ds(0, NPTA - 512), :],
                acc.at[pl.ds(nbase + 512, NPTA - 512), :], sem0)
            for k in range(4):
                pltpu.make_async_copy(
                    rows0, acc.at[pl.ds(nbase, 128), :], sem0).wait()
            pltpu.make_async_copy(
                rows0.at[pl.ds(0, NPTA - 512), :],
                acc.at[pl.ds(nbase + 512, NPTA - 512), :], sem0).wait()
            plsc.subcore_barrier()

            def slab_body(s, _):
                @pl.when(s < nslabs)
                def _():
                    sbase = ebase + s * SLAB
                    pltpu.sync_copy(src_h.at[pl.ds(sbase, SLAB)], isrc)
                    pltpu.sync_copy(dst_h.at[pl.ds(sbase, SLAB)], idst)
                    pltpu.sync_copy(norm_h.at[pl.ds(sbase, SLAB)],
                                    nrm.at[pl.ds(0, SLAB)])

                    build_gidx(0, 0, coff)
                    gather(0).start()

                    def pair_body(b2, _):
                        for db in range(2):
                            b = b2 * 2 + db
                            rbuf = rows[db]

                            @pl.when(b + 1 < nbs)
                            def _():
                                build_gidx(b + 1, 1 - db, coff)
                                gather(1 - db).start()
                            gather(db).wait()

                            def scale_grp(rg, _):
                                nv = nrm[pl.ds(b * B + rg * 16, 16)]
                                for l in range(16):
                                    r = rg * 16 + l
                                    s_ = nv[l]
                                    for j in range(8):
                                        sl = pl.ds(j * 16, 16)
                                        rbuf[r, sl] = rbuf[r, sl] * s_
                                return 0
                            lax.fori_loop(0, B // 16, scale_grp, 0)

                            for j in range(8):
                                idr[db, pl.ds(j * 16, 16)] = (
                                    idst[pl.ds(b * B + j * 16, 16)])
                            pltpu.sync_copy(rbuf, acc.at[idr.at[db]], add=True)
                        return 0
                    lax.fori_loop(0, nbs // 2, pair_body, 0)
                return 0
            lax.fori_loop(0, 2, slab_body, 0)
            plsc.subcore_barrier()

            pltpu.sync_copy(
                acc.at[pl.ds(nbase, NPTA), :],
                out_h.at[cid, pl.ds(nbase, NPTA), pl.ds(c * 128, 128)])
            return 0
        lax.fori_loop(0, C, chunk_body, 0)

    return kern(h_flat, src2, dst2, norm)


def _mm_in_agg(a, W, b):
    K = a.shape[2]
    M = W.shape[1]

    def body(a_ref, w_ref, b_ref, o_ref):
        x = a_ref[0] + a_ref[1]
        y = jnp.dot(x, w_ref[...], preferred_element_type=jnp.float32)
        o_ref[...] = jnp.maximum(y + b_ref[...], 0.0)

    return pl.pallas_call(
        body,
        grid=(N // BN,),
        in_specs=[
            pl.BlockSpec((2, BN, K), lambda i: (0, i, 0)),
            pl.BlockSpec((K, M), lambda i: (0, 0)),
            pl.BlockSpec((1, M), lambda i: (0, 0)),
        ],
        out_specs=pl.BlockSpec((BN, M), lambda i: (i, 0)),
        out_shape=jax.ShapeDtypeStruct((N, M), jnp.float32),
    )(a, W, b.reshape(1, M))


def _mm_chunks(h, W, C):
    K = h.shape[1]

    def body(h_ref, w_ref, o_ref):
        y = jnp.dot(h_ref[...], w_ref[...], preferred_element_type=jnp.float32)
        for j in range(C):
            o_ref[j] = y[:, j * 128:(j + 1) * 128]

    return pl.pallas_call(
        body,
        grid=(N // BN,),
        in_specs=[
            pl.BlockSpec((BN, K), lambda i: (i, 0)),
            pl.BlockSpec((K, C * 128), lambda i: (0, 0)),
        ],
        out_specs=pl.BlockSpec((C, BN, 128), lambda i: (0, i, 0)),
        out_shape=jax.ShapeDtypeStruct((C, N, 128), jnp.float32),
    )(h, W)


def _mm_relu_chunks(p, b_in, W, C):
    K = p.shape[2]

    def body(p_ref, b_ref, w_ref, o_ref):
        x = jnp.maximum(p_ref[0] + p_ref[1] + b_ref[...], 0.0)
        y = jnp.dot(x, w_ref[...], preferred_element_type=jnp.float32)
        for j in range(C):
            o_ref[j] = y[:, j * 128:(j + 1) * 128]

    return pl.pallas_call(
        body,
        grid=(N // BN,),
        in_specs=[
            pl.BlockSpec((2, BN, K), lambda i: (0, i, 0)),
            pl.BlockSpec((1, K), lambda i: (0, 0)),
            pl.BlockSpec((K, C * 128), lambda i: (0, 0)),
        ],
        out_specs=pl.BlockSpec((C, BN, 128), lambda i: (0, i, 0)),
        out_shape=jax.ShapeDtypeStruct((C, N, 128), jnp.float32),
    )(p, b_in.reshape(1, K), W)


def _mm_final(p, b_in, W, b_out):
    K = p.shape[2]
    M = W.shape[1]

    def body(p_ref, bi_ref, w_ref, bo_ref, o_ref):
        x = jnp.maximum(p_ref[0] + p_ref[1] + bi_ref[...], 0.0)
        y = jnp.dot(x, w_ref[...], preferred_element_type=jnp.float32)
        o_ref[...] = y + bo_ref[...]

    return pl.pallas_call(
        body,
        grid=(N // BN,),
        in_specs=[
            pl.BlockSpec((2, BN, K), lambda i: (0, i, 0)),
            pl.BlockSpec((1, K), lambda i: (0, 0)),
            pl.BlockSpec((K, M), lambda i: (0, 0)),
            pl.BlockSpec((1, M), lambda i: (0, 0)),
        ],
        out_specs=pl.BlockSpec((BN, M), lambda i: (i, 0)),
        out_shape=jax.ShapeDtypeStruct((N, M), jnp.float32),
    )(p, b_in.reshape(1, K), W, b_out.reshape(1, M))


def kernel(x, edge_index, edge_attr, W1, b1, W2, b2, W3, b3, W4, b4, W5, b5, Wc, bc):
    src = edge_index[0].astype(jnp.int32)
    dst = edge_index[1].astype(jnp.int32)
    loop_idx = jnp.arange(N, dtype=jnp.int32)
    pad = EPAD - (E + N)
    src_f = jnp.concatenate([src, loop_idx, jnp.zeros((pad,), jnp.int32)])
    dst_f = jnp.concatenate([dst, loop_idx, jnp.zeros((pad,), jnp.int32)])
    w_f = jnp.concatenate([
        edge_attr.astype(jnp.float32),
        jnp.ones((N,), jnp.float32),
        jnp.zeros((pad,), jnp.float32),
    ])

    deg2 = _degrees(dst_f, w_f, jnp.zeros((NPAD,), jnp.float32))
    dinv = _rsqrt_tc(deg2)
    norm = _edge_norm(src_f, dst_f, w_f, dinv)

    a1 = _aggregate(x, src_f, dst_f, norm, 1)
    h1 = _mm_in_agg(a1, W1, b1)

    t2 = _mm_chunks(h1, W2, 8)
    a2 = _aggregate(t2.reshape(8 * N, 128), src_f, dst_f, norm, 8)

    t3 = _mm_relu_chunks(a2, b2, W3, 4)
    a3 = _aggregate(t3.reshape(4 * N, 128), src_f, dst_f, norm, 4)

    t4 = _mm_relu_chunks(a3, b3, W4, 1)
    a4 = _aggregate(t4.reshape(N, 128), src_f, dst_f, norm, 1)

    W5p = jnp.pad(W5, ((0, 0), (0, 64)))
    b5p = jnp.pad(b5, (0, 64))
    t5 = _mm_relu_chunks(a4, b4, W5p, 1)
    a5 = _aggregate(t5.reshape(N, 128), src_f, dst_f, norm, 1)

    Wcp = jnp.pad(Wc, ((0, 64), (0, 125)))
    bcp = jnp.pad(bc, (0, 125))
    out = _mm_final(a5, b5p, Wcp, bcp)
    return out[:, :3]


# speedup vs baseline: 1.0805x; 1.0194x over previous
import functools

import jax
import jax.numpy as jnp
from jax import lax
from jax.experimental import pallas as pl
from jax.experimental.pallas import tpu as pltpu
from jax.experimental.pallas import tpu_sc as plsc

N = 10000
E = 160000
EPAD = 172032
NPAD = 10240
NAGG = 10112
B = 128
BN = 1000
NPTP = NPAD // 16
NPTA = NAGG // 16


def _sc_mesh():
    return plsc.VectorSubcoreMesh(core_axis_name="c", subcore_axis_name="s")


def _degrees(dst, w, zeros1):
    eps = EPAD // 2
    ept = eps // 16
    nit = ept // B

    @functools.partial(
        pl.kernel,
        mesh=_sc_mesh(),
        out_type=jax.ShapeDtypeStruct((2, NPAD), jnp.float32),
        scratch_types=[
            pltpu.VMEM((ept,), jnp.int32),
            pltpu.VMEM((ept,), jnp.float32),
            pltpu.VMEM((ept // B, 128), jnp.int32),
            pltpu.VMEM_SHARED((NPAD,), jnp.float32),
            pltpu.SemaphoreType.DMA,
        ],
    )
    def kern(dst_h, w_h, z_h, deg_h, dflat, wflat, idx2, deg_sh, sem):
        cid = lax.axis_index("c")
        sid = lax.axis_index("s")

        nsl = pl.ds(sid * NPTP, NPTP)
        pltpu.sync_copy(z_h.at[nsl], deg_sh.at[nsl])

        ebase = cid * eps + sid * ept
        pltpu.sync_copy(dst_h.at[pl.ds(ebase, ept)], dflat)
        pltpu.sync_copy(w_h.at[pl.ds(ebase, ept)], wflat)

        def expand(i, _):
            for j in range(8):
                idx2[i, pl.ds(j * 16, 16)] = dflat[pl.ds(i * B + j * 16, 16)]
            return 0
        lax.fori_loop(0, nit, expand, 0)
        plsc.subcore_barrier()

        def fire(b, _):
            pltpu.async_copy(
                wflat.at[pl.ds(b * B, B)], deg_sh.at[idx2.at[b]], sem, add=True)
            return 0
        lax.fori_loop(0, nit, fire, 0)

        def drain(b, _):
            pltpu.make_async_copy(
                wflat.at[pl.ds(b * B, B)], deg_sh.at[idx2.at[b]], sem).wait()
            return 0
        lax.fori_loop(0, nit, drain, 0)
        plsc.subcore_barrier()

        pltpu.sync_copy(deg_sh.at[nsl], deg_h.at[cid, nsl])

    return kern(dst, w, zeros1)


def _rsqrt_tc(deg2):
    def body(d_ref, o_ref):
        d = d_ref[0] + d_ref[1]
        o_ref[...] = lax.rsqrt(jnp.maximum(d, 1e-12)).reshape(1, NPAD)

    return pl.pallas_call(
        body,
        grid=(1,),
        in_specs=[pl.BlockSpec((2, NPAD), lambda i: (0, 0))],
        out_specs=pl.BlockSpec((1, NPAD), lambda i: (0, 0)),
        out_shape=jax.ShapeDtypeStruct((1, NPAD), jnp.float32),
    )(deg2).reshape(NPAD)


def _edge_norm(src, dst, w, dinv):
    eps = EPAD // 2
    ept = eps // 16
    nit = ept // B

    @functools.partial(
        pl.kernel,
        mesh=_sc_mesh(),
        out_type=jax.ShapeDtypeStruct((EPAD,), jnp.float32),
        scratch_types=[
            pltpu.VMEM((ept,), jnp.int32),
            pltpu.VMEM((ept,), jnp.int32),
            pltpu.VMEM((ept,), jnp.float32),
            pltpu.VMEM((ept,), jnp.float32),
            pltpu.VMEM((ept,), jnp.float32),
            pltpu.SemaphoreType.DMA,
        ],
    )
    def kern(src_h, dst_h, w_h, dinv_h, norm_h, sflat, dflat, wflat, dsv, ddv, sem):
        cid = lax.axis_index("c")
        sid = lax.axis_index("s")
        ebase = cid * eps + sid * ept

        pltpu.sync_copy(src_h.at[pl.ds(ebase, ept)], sflat)
        pltpu.sync_copy(dst_h.at[pl.ds(ebase, ept)], dflat)
        pltpu.sync_copy(w_h.at[pl.ds(ebase, ept)], wflat)

        def fire(b, _):
            sl = pl.ds(b * B, B)
            pltpu.async_copy(dinv_h.at[sflat.at[sl]], dsv.at[sl], sem)
            pltpu.async_copy(dinv_h.at[dflat.at[sl]], ddv.at[sl], sem)
            return 0
        lax.fori_loop(0, nit, fire, 0)

        def drain(b, _):
            sl = pl.ds(b * B, B)
            pltpu.make_async_copy(dinv_h.at[sflat.at[sl]], dsv.at[sl], sem).wait()
            pltpu.make_async_copy(dinv_h.at[dflat.at[sl]], ddv.at[sl], sem).wait()
            return 0
        lax.fori_loop(0, nit, drain, 0)

        def grp(g, _):
            sl = pl.ds(g * 16, 16)
            wflat[sl] = wflat[sl] * dsv[sl] * ddv[sl]
            return 0
        lax.fori_loop(0, ept // 16, grp, 0)
        pltpu.sync_copy(wflat, norm_h.at[pl.ds(ebase, ept)])

    return kern(src, dst, w, dinv)


def _aggregate(h_flat, src2, dst2, norm, C, fastc=0):
    SLAB = 3584
    nbs = SLAB // B

    @functools.partial(
        pl.kernel,
        mesh=_sc_mesh(),
        out_type=jax.ShapeDtypeStruct((2, NAGG, C * 128), jnp.float32),
        scratch_types=[
            pltpu.VMEM((SLAB,), jnp.int32),
            pltpu.VMEM((SLAB,), jnp.int32),
            pltpu.VMEM((2, 128), jnp.int32),
            pltpu.VMEM((2, 128), jnp.int32),
            pltpu.VMEM((SLAB + 16,), jnp.float32),
            pltpu.VMEM((B, 128), jnp.float32),
            pltpu.VMEM((B, 128), jnp.float32),
            pltpu.VMEM_SHARED((NAGG, 128), jnp.float32),
            pltpu.SemaphoreType.DMA,
            pltpu.SemaphoreType.DMA,
            pltpu.SemaphoreType.DMA,
            pltpu.SemaphoreType.DMA,
        ],
    )
    def kern(h_h, src_h, dst_h, norm_h, out_h,
             isrc, idst, ish, idr, nrm, rows0, rows1, acc,
             sem0, sem1, ssem0, ssem1):
        cid = lax.axis_index("c")
        sid = lax.axis_index("s")
        nslabs = jnp.where(cid == fastc, 2, 1)
        ebase = jnp.where(cid == fastc, 0, 32 * SLAB) + sid * (nslabs * SLAB)
        nbase = sid * NPTA

        rows = (rows0, rows1)
        sems = (sem0, sem1)
        ssems = (ssem0, ssem1)

        def gather(db):
            return pltpu.make_async_copy(h_h.at[ish.at[db]], rows[db], sems[db])

        def scatter(db):
            return pltpu.make_async_copy(
                rows[db], acc.at[idr.at[db]], ssems[db])

        def build_gidx(b, ring, coff):
            for j in range(8):
                ish[ring, pl.ds(j * 16, 16)] = (
                    isrc[pl.ds(b * B + j * 16, 16)] + coff)

        def chunk_body(c, _):
            coff = c * N

            def zs(i, _):
                for j in range(8):
                    rows0[i, pl.ds(j * 16, 16)] = jnp.zeros((16,), jnp.float32)
                return 0
            lax.fori_loop(0, B, zs, 0)
            for k in range(4):
                pltpu.async_copy(
                    rows0, acc.at[pl.ds(nbase + k * 128, 128), :], sem0)
            pltpu.async_copy(
                rows0.at[pl.ds(0, NPTA - 512), :],
                acc.at[pl.ds(nbase + 512, NPTA - 512), :], sem0)
            for k in range(4):
                pltpu.make_async_copy(
                    rows0, acc.at[pl.ds(nbase, 128), :], sem0).wait()
            pltpu.make_async_copy(
                rows0.at[pl.ds(0, NPTA - 512), :],
                acc.at[pl.ds(nbase + 512, NPTA - 512), :], sem0).wait()
            plsc.subcore_barrier()

            def slab_body(s, _):
                @pl.when(s < nslabs)
                def _():
                    sbase = ebase + s * SLAB
                    pltpu.sync_copy(src_h.at[pl.ds(sbase, SLAB)], isrc)
                    pltpu.sync_copy(dst_h.at[pl.ds(sbase, SLAB)], idst)
                    pltpu.sync_copy(norm_h.at[pl.ds(sbase, SLAB)],
                                    nrm.at[pl.ds(0, SLAB)])

                    build_gidx(0, 0, coff)
                    gather(0).start()

                    def pair_body(b2, _):
                        for db in range(2):
                            b = b2 * 2 + db
                            rbuf = rows[db]

                            @pl.when(b + 1 < nbs)
                            def _():
                                build_gidx(b + 1, 1 - db, coff)
                                gather(1 - db).start()
                            gather(db).wait()

                            def scale_grp(rg, _):
                                nv = nrm[pl.ds(b * B + rg * 16, 16)]
                                for l in range(16):
                                    r = rg * 16 + l
                                    s_ = nv[l]
                                    for j in range(8):
                                        sl = pl.ds(j * 16, 16)
                                        rbuf[r, sl] = rbuf[r, sl] * s_
                                return 0
                            lax.fori_loop(0, B // 16, scale_grp, 0)

                            for j in range(8):
                                idr[db, pl.ds(j * 16, 16)] = (
                                    idst[pl.ds(b * B + j * 16, 16)])
                            pltpu.sync_copy(rbuf, acc.at[idr.at[db]], add=True)
                        return 0
                    lax.fori_loop(0, nbs // 2, pair_body, 0)
                return 0
            lax.fori_loop(0, 2, slab_body, 0)
            plsc.subcore_barrier()

            pltpu.sync_copy(
                acc.at[pl.ds(nbase, NPTA), :],
                out_h.at[cid, pl.ds(nbase, NPTA), pl.ds(c * 128, 128)])
            return 0
        lax.fori_loop(0, C, chunk_body, 0)

    return kern(h_flat, src2, dst2, norm)


def _mm_in_agg(a, W, b):
    K = a.shape[2]
    M = W.shape[1]

    def body(a_ref, w_ref, b_ref, o_ref):
        x = a_ref[0] + a_ref[1]
        y = jnp.dot(x, w_ref[...], preferred_element_type=jnp.float32)
        o_ref[...] = jnp.maximum(y + b_ref[...], 0.0)

    return pl.pallas_call(
        body,
        grid=(N // BN,),
        in_specs=[
            pl.BlockSpec((2, BN, K), lambda i: (0, i, 0)),
            pl.BlockSpec((K, M), lambda i: (0, 0)),
            pl.BlockSpec((1, M), lambda i: (0, 0)),
        ],
        out_specs=pl.BlockSpec((BN, M), lambda i: (i, 0)),
        out_shape=jax.ShapeDtypeStruct((N, M), jnp.float32),
    )(a, W, b.reshape(1, M))


def _mm_fused12(a, W1, b1, W2, C):
    K = a.shape[2]
    M = W1.shape[1]

    def body(a_ref, w1_ref, b1_ref, w2_ref, o_ref):
        x = a_ref[0] + a_ref[1]
        y1 = jnp.dot(x, w1_ref[...], preferred_element_type=jnp.float32)
        y1 = jnp.maximum(y1 + b1_ref[...], 0.0)
        y = jnp.dot(y1, w2_ref[...], preferred_element_type=jnp.float32)
        for j in range(C):
            o_ref[j] = y[:, j * 128:(j + 1) * 128]

    return pl.pallas_call(
        body,
        grid=(N // BN,),
        in_specs=[
            pl.BlockSpec((2, BN, K), lambda i: (0, i, 0)),
            pl.BlockSpec((K, M), lambda i: (0, 0)),
            pl.BlockSpec((1, M), lambda i: (0, 0)),
            pl.BlockSpec((M, C * 128), lambda i: (0, 0)),
        ],
        out_specs=pl.BlockSpec((C, BN, 128), lambda i: (0, i, 0)),
        out_shape=jax.ShapeDtypeStruct((C, N, 128), jnp.float32),
    )(a, W1, b1.reshape(1, M), W2)


def _mm_relu_chunks(p, b_in, W, C):
    K = p.shape[2]

    def body(p_ref, b_ref, w_ref, o_ref):
        x = jnp.maximum(p_ref[0] + p_ref[1] + b_ref[...], 0.0)
        y = jnp.dot(x, w_ref[...], preferred_element_type=jnp.float32)
        for j in range(C):
            o_ref[j] = y[:, j * 128:(j + 1) * 128]

    return pl.pallas_call(
        body,
        grid=(N // BN,),
        in_specs=[
            pl.BlockSpec((2, BN, K), lambda i: (0, i, 0)),
            pl.BlockSpec((1, K), lambda i: (0, 0)),
            pl.BlockSpec((K, C * 128), lambda i: (0, 0)),
        ],
        out_specs=pl.BlockSpec((C, BN, 128), lambda i: (0, i, 0)),
        out_shape=jax.ShapeDtypeStruct((C, N, 128), jnp.float32),
    )(p, b_in.reshape(1, K), W)


def _mm_final(p, b_in, W, b_out):
    K = p.shape[2]
    M = W.shape[1]

    def body(p_ref, bi_ref, w_ref, bo_ref, o_ref):
        x = jnp.maximum(p_ref[0] + p_ref[1] + bi_ref[...], 0.0)
        y = jnp.dot(x, w_ref[...], preferred_element_type=jnp.float32)
        o_ref[...] = y + bo_ref[...]

    return pl.pallas_call(
        body,
        grid=(N // BN,),
        in_specs=[
            pl.BlockSpec((2, BN, K), lambda i: (0, i, 0)),
            pl.BlockSpec((1, K), lambda i: (0, 0)),
            pl.BlockSpec((K, M), lambda i: (0, 0)),
            pl.BlockSpec((1, M), lambda i: (0, 0)),
        ],
        out_specs=pl.BlockSpec((BN, M), lambda i: (i, 0)),
        out_shape=jax.ShapeDtypeStruct((N, M), jnp.float32),
    )(p, b_in.reshape(1, K), W, b_out.reshape(1, M))


def kernel(x, edge_index, edge_attr, W1, b1, W2, b2, W3, b3, W4, b4, W5, b5, Wc, bc):
    src = edge_index[0].astype(jnp.int32)
    dst = edge_index[1].astype(jnp.int32)
    loop_idx = jnp.arange(N, dtype=jnp.int32)
    pad = EPAD - (E + N)
    src_f = jnp.concatenate([src, loop_idx, jnp.zeros((pad,), jnp.int32)])
    dst_f = jnp.concatenate([dst, loop_idx, jnp.zeros((pad,), jnp.int32)])
    w_f = jnp.concatenate([
        edge_attr.astype(jnp.float32),
        jnp.ones((N,), jnp.float32),
        jnp.zeros((pad,), jnp.float32),
    ])

    deg2 = _degrees(dst_f, w_f, jnp.zeros((NPAD,), jnp.float32))
    dinv = _rsqrt_tc(deg2)
    norm = _edge_norm(src_f, dst_f, w_f, dinv)

    a1 = _aggregate(x, src_f, dst_f, norm, 1)
    t2 = _mm_fused12(a1, W1, b1, W2, 8)
    a2 = _aggregate(t2.reshape(8 * N, 128), src_f, dst_f, norm, 8)

    t3 = _mm_relu_chunks(a2, b2, W3, 4)
    a3 = _aggregate(t3.reshape(4 * N, 128), src_f, dst_f, norm, 4)

    t4 = _mm_relu_chunks(a3, b3, W4, 1)
    a4 = _aggregate(t4.reshape(N, 128), src_f, dst_f, norm, 1)

    W5p = jnp.pad(W5, ((0, 0), (0, 64)))
    b5p = jnp.pad(b5, (0, 64))
    t5 = _mm_relu_chunks(a4, b4, W5p, 1)
    a5 = _aggregate(t5.reshape(N, 128), src_f, dst_f, norm, 1)

    Wcp = jnp.pad(Wc, ((0, 64), (0, 125)))
    bcp = jnp.pad(bc, (0, 125))
    out = _mm_final(a5, b5p, Wcp, bcp)
    return out[:, :3]
